# Initial kernel scaffold; baseline (speedup 1.0000x reference)
import jax
import jax.numpy as jnp
from jax.experimental import pallas as pl


def kernel(x, edge_index, W1, b1, c1, W2, b2, c2, W3, b3, W4, b4):
    raise NotImplementedError("write your pallas kernel here")


# speedup vs baseline: 4.9615x; 4.9615x over previous
import functools

import jax
import jax.numpy as jnp
from jax import lax
from jax.experimental import pallas as pl
from jax.experimental.pallas import tpu as pltpu
from jax.experimental.pallas import tpu_sc as plsc

N = 10000
E = 320000
D = 128
D2 = 256
NC, NS = 2, 16
NW = NC * NS
EPW = E // NW
K = 128
CH = 80
EPW_P = CH * K
EP = NW * EPW_P
AGG = 10112
STRIPE = AGG // NS

_mesh = functools.partial(
    plsc.VectorSubcoreMesh, core_axis_name="c", subcore_axis_name="s")


def _sc_histogram(colp, zeros1, ones):
    @functools.partial(
        pl.kernel,
        out_type=jax.ShapeDtypeStruct((NC, AGG, 1), jnp.float32),
        mesh=_mesh(),
        scratch_types=[
            pltpu.VMEM((CH, K), jnp.int32),
            pltpu.VMEM((K, 1), jnp.float32),
            pltpu.VMEM_SHARED((AGG, 1), jnp.float32),
        ],
    )
    def run(colp_hbm, z1_hbm, ones_hbm, degp_hbm, col_v, ones_v, deg_sh):
        c = lax.axis_index("c")
        s = lax.axis_index("s")
        wid = c * NS + s
        pltpu.sync_copy(z1_hbm.at[pl.ds(s * STRIPE, STRIPE)],
                        deg_sh.at[pl.ds(s * STRIPE, STRIPE)])
        pltpu.sync_copy(ones_hbm, ones_v)
        pltpu.sync_copy(colp_hbm.at[wid], col_v)
        plsc.subcore_barrier()

        def step(j, carry):
            pltpu.sync_copy(ones_v, deg_sh.at[col_v.at[j]], add=True)
            return carry

        lax.fori_loop(0, CH, step, 0)
        plsc.subcore_barrier()
        pltpu.sync_copy(deg_sh.at[pl.ds(s * STRIPE, STRIPE)],
                        degp_hbm.at[c, pl.ds(s * STRIPE, STRIPE)])

    return run(colp, zeros1, ones)


def _sc_conv(g, rowp, colp, zeros):
    @functools.partial(
        pl.kernel,
        out_type=jax.ShapeDtypeStruct((NC, AGG, D), jnp.float32),
        mesh=_mesh(),
        scratch_types=[
            pltpu.VMEM((CH, K), jnp.int32),
            pltpu.VMEM((CH, K), jnp.int32),
            pltpu.VMEM((K, D), jnp.float32),
            pltpu.VMEM_SHARED((AGG, D), jnp.float32),
            pltpu.SemaphoreType.DMA,
        ],
    )
    def run(g_hbm, rowp_hbm, colp_hbm, z_hbm, aggp_hbm,
            row_v, col_v, buf, agg_sh, sem):
        c = lax.axis_index("c")
        s = lax.axis_index("s")
        wid = c * NS + s
        pltpu.sync_copy(z_hbm.at[pl.ds(s * STRIPE, STRIPE)],
                        agg_sh.at[pl.ds(s * STRIPE, STRIPE)])
        pltpu.sync_copy(rowp_hbm.at[wid], row_v)
        pltpu.sync_copy(colp_hbm.at[wid], col_v)
        plsc.subcore_barrier()

        def step(j, carry):
            pltpu.async_copy(g_hbm.at[row_v.at[j]], buf, sem).wait()
            pltpu.sync_copy(buf, agg_sh.at[col_v.at[j]], add=True)
            return carry

        lax.fori_loop(0, CH, step, 0)
        plsc.subcore_barrier()
        pltpu.sync_copy(agg_sh.at[pl.ds(s * STRIPE, STRIPE)],
                        aggp_hbm.at[c, pl.ds(s * STRIPE, STRIPE)])

    return run(g, rowp, colp, zeros)


def _sc_edge_emb(h, srcp, dstp):
    @functools.partial(
        pl.kernel,
        out_type=jax.ShapeDtypeStruct((EP, D), jnp.float32),
        mesh=_mesh(),
        scratch_types=[
            pltpu.VMEM((CH, K), jnp.int32),
            pltpu.VMEM((CH, K), jnp.int32),
            pltpu.VMEM((K, D), jnp.float32),
            pltpu.VMEM((K, D), jnp.float32),
            pltpu.VMEM((K,), jnp.int32),
            pltpu.VMEM_SHARED((NS * K, D), jnp.float32),
            pltpu.SemaphoreType.DMA,
            pltpu.SemaphoreType.DMA,
        ],
    )
    def run(h_hbm, srcp_hbm, dstp_hbm, emb_hbm,
            src_v, dst_v, buf_a, buf_b, idx_v, stage_sh, sem_a, sem_b):
        c = lax.axis_index("c")
        s = lax.axis_index("s")
        wid = c * NS + s
        pltpu.sync_copy(srcp_hbm.at[wid], src_v)
        pltpu.sync_copy(dstp_hbm.at[wid], dst_v)
        for t in range(K // 16):
            idx_v[pl.ds(t * 16, 16)] = (
                lax.iota(jnp.int32, 16) + (s * K + t * 16))

        def step(j, carry):
            da = pltpu.async_copy(h_hbm.at[src_v.at[j]], buf_a, sem_a)
            db = pltpu.async_copy(h_hbm.at[dst_v.at[j]], buf_b, sem_b)
            da.wait()
            db.wait()
            pltpu.sync_copy(buf_a, stage_sh.at[pl.ds(s * K, K)])
            pltpu.sync_copy(buf_b, stage_sh.at[idx_v], add=True)
            pltpu.sync_copy(stage_sh.at[pl.ds(s * K, K)],
                            emb_hbm.at[pl.ds(wid * EPW_P + j * K, K)])
            return carry

        lax.fori_loop(0, CH, step, 0)

    return run(h, srcp, dstp)


def _tc_lin1(x, w1t, b1, degp2):
    BR = 1000
    G = N // BR

    def body(x_ref, w_ref, b_ref, d_ref, h_ref, g_ref, dinv_ref, invd_ref):
        h = jnp.dot(x_ref[...], w_ref[...],
                    preferred_element_type=jnp.float32) + b_ref[...]
        d = d_ref[...]
        deg = d[0] + d[1] + 1.0
        dinv = lax.rsqrt(deg)
        h_ref[...] = h
        g_ref[...] = h * dinv
        dinv_ref[...] = dinv
        invd_ref[...] = 1.0 / deg

    return pl.pallas_call(
        body,
        grid=(G,),
        in_specs=[
            pl.BlockSpec((BR, D), lambda r: (r, 0)),
            pl.BlockSpec((D, D), lambda r: (0, 0)),
            pl.BlockSpec((1, D), lambda r: (0, 0)),
            pl.BlockSpec((2, BR, 1), lambda r: (0, r, 0)),
        ],
        out_specs=[
            pl.BlockSpec((BR, D), lambda r: (r, 0)),
            pl.BlockSpec((BR, D), lambda r: (r, 0)),
            pl.BlockSpec((BR, 1), lambda r: (r, 0)),
            pl.BlockSpec((BR, 1), lambda r: (r, 0)),
        ],
        out_shape=[
            jax.ShapeDtypeStruct((N, D), jnp.float32),
            jax.ShapeDtypeStruct((N, D), jnp.float32),
            jax.ShapeDtypeStruct((N, 1), jnp.float32),
            jax.ShapeDtypeStruct((N, 1), jnp.float32),
        ],
    )(x, w1t, b1, degp2)


def _tc_mid(h0, a0, a1, dinv, invd, w2t, b2, c1):
    BR = 1000
    G = N // BR

    def body(h_ref, a0_ref, a1_ref, dinv_ref, invd_ref, w_ref, b_ref,
             c_ref, h2_ref, g1_ref):
        h = h_ref[...]
        dinv = dinv_ref[...]
        o = (0.5 * h - dinv * (a0_ref[...] + a1_ref[...])
             - h * invd_ref[...] + c_ref[...])
        o = jnp.where(o > 0, o, jnp.exp(jnp.minimum(o, 0.0)) - 1.0)
        h2 = jnp.dot(o, w_ref[...],
                     preferred_element_type=jnp.float32) + b_ref[...]
        h2_ref[...] = h2
        g1_ref[...] = h2 * dinv

    return pl.pallas_call(
        body,
        grid=(G,),
        in_specs=[
            pl.BlockSpec((BR, D), lambda r: (r, 0)),
            pl.BlockSpec((BR, D), lambda r: (r, 0)),
            pl.BlockSpec((BR, D), lambda r: (r, 0)),
            pl.BlockSpec((BR, 1), lambda r: (r, 0)),
            pl.BlockSpec((BR, 1), lambda r: (r, 0)),
            pl.BlockSpec((D, D), lambda r: (0, 0)),
            pl.BlockSpec((1, D), lambda r: (0, 0)),
            pl.BlockSpec((1, D), lambda r: (0, 0)),
        ],
        out_specs=[
            pl.BlockSpec((BR, D), lambda r: (r, 0)),
            pl.BlockSpec((BR, D), lambda r: (r, 0)),
        ],
        out_shape=[
            jax.ShapeDtypeStruct((N, D), jnp.float32),
            jax.ShapeDtypeStruct((N, D), jnp.float32),
        ],
    )(h0, a0, a1, dinv, invd, w2t, b2, c1)


def _tc_combine2(h2, a0, a1, dinv, invd, c2):
    BR = 1000
    G = N // BR

    def body(h_ref, a0_ref, a1_ref, dinv_ref, invd_ref, c_ref, o_ref):
        h = h_ref[...]
        o_ref[...] = (0.5 * h - dinv_ref[...] * (a0_ref[...] + a1_ref[...])
                      - h * invd_ref[...] + c_ref[...])

    return pl.pallas_call(
        body,
        grid=(G,),
        in_specs=[
            pl.BlockSpec((BR, D), lambda r: (r, 0)),
            pl.BlockSpec((BR, D), lambda r: (r, 0)),
            pl.BlockSpec((BR, D), lambda r: (r, 0)),
            pl.BlockSpec((BR, 1), lambda r: (r, 0)),
            pl.BlockSpec((BR, 1), lambda r: (r, 0)),
            pl.BlockSpec((1, D), lambda r: (0, 0)),
        ],
        out_specs=[pl.BlockSpec((BR, D), lambda r: (r, 0))],
        out_shape=[jax.ShapeDtypeStruct((N, D), jnp.float32)],
    )(h2, a0, a1, dinv, invd, c2)[0]


def _tc_edge_mlp(emb, w3t, b3, w4c, b4):
    BE = 1024
    G = EP // BE

    def body(e_ref, w3_ref, b3_ref, w4_ref, b4_ref, o_ref):
        hdd = jnp.maximum(
            jnp.dot(e_ref[...], w3_ref[...],
                    preferred_element_type=jnp.float32) + b3_ref[...], 0.0)
        z = jnp.dot(hdd, w4_ref[...],
                    preferred_element_type=jnp.float32) + b4_ref[...]
        o_ref[...] = 1.0 / (1.0 + jnp.exp(-z))

    return pl.pallas_call(
        body,
        grid=(G,),
        in_specs=[
            pl.BlockSpec((BE, D), lambda r: (r, 0)),
            pl.BlockSpec((D, D2), lambda r: (0, 0)),
            pl.BlockSpec((1, D2), lambda r: (0, 0)),
            pl.BlockSpec((D2, 1), lambda r: (0, 0)),
            pl.BlockSpec((1, 1), lambda r: (0, 0)),
        ],
        out_specs=[pl.BlockSpec((BE, 1), lambda r: (r, 0))],
        out_shape=[jax.ShapeDtypeStruct((EP, 1), jnp.float32)],
    )(emb, w3t, b3, w4c, b4)[0]


def kernel(x, edge_index, W1, b1, c1, W2, b2, c2, W3, b3, W4, b4):
    f32 = jnp.float32
    i32 = jnp.int32
    row = edge_index[0]
    col = edge_index[1]

    pad_n = EPW_P - EPW
    rowp = jnp.concatenate(
        [row.reshape(NW, EPW), jnp.zeros((NW, pad_n), i32)],
        axis=1).reshape(NW, CH, K)
    colp = jnp.concatenate(
        [col.reshape(NW, EPW), jnp.full((NW, pad_n), N, i32)],
        axis=1).reshape(NW, CH, K)
    dstp0 = jnp.concatenate(
        [col.reshape(NW, EPW), jnp.zeros((NW, pad_n), i32)],
        axis=1).reshape(NW, CH, K)

    zeros = jnp.zeros((AGG, D), f32)
    zeros1 = jnp.zeros((AGG, 1), f32)
    ones = jnp.ones((K, 1), f32)

    w1t = W1.T
    w2t = W2.T
    w3t = W3.T
    w4c = W4.T
    b1r = b1.reshape(1, D)
    b2r = b2.reshape(1, D)
    b3r = b3.reshape(1, D2)
    b4r = b4.reshape(1, 1)
    c1r = c1.reshape(1, D)
    c2r = c2.reshape(1, D)

    degp = _sc_histogram(colp, zeros1, ones)
    degp2 = degp[:, :N, :]

    h0, g0, dinv, invd = _tc_lin1(x, w1t, b1r, degp2)

    agg0 = _sc_conv(g0, rowp, colp, zeros)
    h2, g1 = _tc_mid(h0, agg0[0, :N], agg0[1, :N], dinv, invd,
                     w2t, b2r, c1r)

    agg1 = _sc_conv(g1, rowp, colp, zeros)
    hconv = _tc_combine2(h2, agg1[0, :N], agg1[1, :N], dinv, invd, c2r)

    emb = _sc_edge_emb(hconv, rowp, dstp0)
    out = _tc_edge_mlp(emb, w3t, b3r, w4c, b4r)

    return out.reshape(NW, EPW_P)[:, :EPW].reshape(E, 1)
